# Initial kernel scaffold; baseline (speedup 1.0000x reference)
import jax
import jax.numpy as jnp
from jax.experimental import pallas as pl


def kernel(x, hyperedge_index, W):
    raise NotImplementedError("write your pallas kernel here")


# speedup vs baseline: 11.5666x; 11.5666x over previous
import jax
import jax.numpy as jnp
from jax import lax
from jax.experimental import pallas as pl
from jax.experimental.pallas import tpu as pltpu
from jax.experimental.pallas import tpu_sc as plsc
from jax._src.pallas import mpmd as _mpmd
from jax._src.pallas import core as _pallas_core

N = 10000
NP = 10240
E = 320000
D = 128
NC = 2
NS = 16
NW = NC * NS
T = E // NW
K = 80
CH = T // K
RPW = NP // NS
L = 16
NB = 8
DB = NP // 1280

_vmesh = plsc.VectorSubcoreMesh(core_axis_name="core", subcore_axis_name="subcore")
_smesh = plsc.ScalarSubcoreMesh(axis_name="core", num_cores=NC)


def _tec_vmem(shape, dtype):
    return _pallas_core.CoreMemorySpace(pltpu.VMEM, _vmesh)(shape, dtype)


def _deg_tec(he_hbm, out_hbm, idx_v, hist_v, cnt_v):
    c = lax.axis_index("core")
    s = lax.axis_index("subcore")
    wid = s * NC + c

    iota = lax.iota(jnp.int32, L)
    bank = lax.bitwise_and(iota, NB - 1)
    m_lo = iota < NB
    m_hi = jnp.logical_not(m_lo)
    ones = jnp.ones((L,), jnp.float32)

    for which in range(2):
        pltpu.sync_copy(he_hbm.at[which, wid], idx_v)

        @pl.loop(0, NP * NB, step=L)
        def _(i):
            hist_v[pl.ds(i, L)] = jnp.zeros((L,), jnp.float32)

        @pl.loop(0, CH)
        def _(j):
            @pl.loop(0, K, step=L)
            def _(m):
                v = idx_v[j, pl.ds(m, L)]
                flat = v * NB + bank
                plsc.addupdate_scatter(hist_v, [flat], ones, mask=m_lo)
                plsc.addupdate_scatter(hist_v, [flat], ones, mask=m_hi)

        @pl.loop(0, NP, step=L)
        def _(b):
            base = (b + iota) * NB
            acc = plsc.load_gather(hist_v, [base])
            for q in range(1, NB):
                acc = acc + plsc.load_gather(hist_v, [base + q])
            cnt_v[pl.ds(b, L)] = acc

        @pl.loop(0, DB)
        def _(t):
            pltpu.sync_copy(cnt_v.at[pl.ds(t * 1280, 1280)],
                            out_hbm.at[t, wid, which])


_degrees_sc = pl.kernel(
    _deg_tec,
    out_type=jax.ShapeDtypeStruct((DB, NW, 2, 1280), jnp.float32),
    mesh=_vmesh,
    compiler_params=pltpu.CompilerParams(needs_layout_passes=False),
    scratch_types=[
        pltpu.VMEM((CH, K), jnp.int32),
        pltpu.VMEM((NP * NB,), jnp.float32),
        pltpu.VMEM((NP,), jnp.float32),
    ],
)


def _agg_scs(*_refs):
    pass


def _agg_tec(scaled0_hbm, scaled1_hbm, he_hbm, out_hbm, idx0_v, idx1_v,
             rows_v, zero_v, acc_sh):
    pl.run_scoped(
        lambda sem0, sem1: _agg_tec_inner(scaled0_hbm, scaled1_hbm, he_hbm,
                                          out_hbm, idx0_v, idx1_v, rows_v,
                                          zero_v, acc_sh, sem0, sem1),
        pltpu.SemaphoreType.DMA,
        pltpu.SemaphoreType.DMA,
    )


def _agg_tec_inner(scaled0_hbm, scaled1_hbm, he_hbm, out_hbm, idx0_v, idx1_v,
                   rows_v, zero_v, acc_sh, sem0, sem1):
    c = lax.axis_index("core")
    s = lax.axis_index("subcore")
    wid = s * NC + c
    zr = RPW // 5
    HD = D // 2

    @pl.loop(0, zr)
    def _(i):
        @pl.loop(0, HD, step=L)
        def _(j):
            zero_v[i, pl.ds(j, L)] = jnp.zeros((L,), jnp.float32)

    pltpu.sync_copy(he_hbm.at[0, wid], idx0_v)
    pltpu.sync_copy(he_hbm.at[1, wid], idx1_v)

    for h in range(2):
        src_hbm = scaled0_hbm if h == 0 else scaled1_hbm

        @pl.loop(0, 5)
        def _(t):
            pltpu.sync_copy(zero_v, acc_sh.at[pl.ds(s * RPW + t * zr, zr)])

        plsc.subcore_barrier()

        @pl.loop(0, CH - 1, step=2)
        def _(j):
            g0 = pltpu.async_copy(src_hbm.at[idx1_v.at[j]], rows_v.at[0],
                                  sem0)
            g1 = pltpu.async_copy(src_hbm.at[idx1_v.at[j + 1]], rows_v.at[1],
                                  sem1)
            g0.wait()
            pltpu.sync_copy(rows_v.at[0], acc_sh.at[idx0_v.at[j]], add=True)
            g1.wait()
            pltpu.sync_copy(rows_v.at[1], acc_sh.at[idx0_v.at[j + 1]],
                            add=True)

        last = CH - 1
        pltpu.async_copy(src_hbm.at[idx1_v.at[last]], rows_v.at[0],
                         sem0).wait()
        pltpu.sync_copy(rows_v.at[0], acc_sh.at[idx0_v.at[last]], add=True)

        plsc.subcore_barrier()
        pltpu.sync_copy(acc_sh.at[pl.ds(s * RPW, RPW)],
                        out_hbm.at[c, h, pl.ds(s * RPW, RPW)])
        plsc.subcore_barrier()


_aggregate_sc = _mpmd.mpmd_map(
    [(_smesh, _agg_scs), (_vmesh, _agg_tec)],
    out_types=[jax.ShapeDtypeStruct((NC, 2, NP, D // 2), jnp.float32)],
    scratch_types=[
        _tec_vmem((CH, K), jnp.int32),
        _tec_vmem((CH, K), jnp.int32),
        _tec_vmem((2, K, D // 2), jnp.float32),
        _tec_vmem((RPW // 5, D // 2), jnp.float32),
        pltpu.VMEM_SHARED((NP, D // 2), jnp.float32),
    ],
    compiler_params=pltpu.CompilerParams(use_tc_tiling_on_sc=False),
)


_BM = 1280


def _matmul_scale_body(x_ref, w_ref, degp_ref, o0_ref, o1_ref):
    d = jnp.sum(degp_ref[0][:, 1, :], axis=0)
    b = 1.0 / (d + 1e-5)
    bcol = jnp.broadcast_to(b.reshape(_BM, 1), (_BM, D // 2))
    acc = jnp.dot(x_ref[...], w_ref[...], preferred_element_type=jnp.float32)
    o0_ref[...] = acc[:, :D // 2] * bcol
    o1_ref[...] = acc[:, D // 2:] * bcol


def _finalize_body(acc_ref, degp_ref, o_ref):
    s0 = acc_ref[0, 0] + acc_ref[1, 0]
    s1 = acc_ref[0, 1] + acc_ref[1, 1]
    ssum = jnp.concatenate([s0, s1], axis=1)
    d = jnp.sum(degp_ref[0][:, 0, :], axis=0)
    a = jnp.where(d > 0, lax.rsqrt(d), 0.0)
    acol = jnp.broadcast_to(a.reshape(_BM, 1), (_BM, D))
    o_ref[...] = ssum * acol


_matmul_scale = pl.pallas_call(
    _matmul_scale_body,
    grid=(NP // _BM,),
    in_specs=[
        pl.BlockSpec((_BM, D), lambda i: (i, 0)),
        pl.BlockSpec((D, D), lambda i: (0, 0)),
        pl.BlockSpec((1, NW, 2, _BM), lambda i: (i, 0, 0, 0)),
    ],
    out_specs=[pl.BlockSpec((_BM, D // 2), lambda i: (i, 0)),
               pl.BlockSpec((_BM, D // 2), lambda i: (i, 0))],
    out_shape=[jax.ShapeDtypeStruct((NP, D // 2), jnp.float32),
               jax.ShapeDtypeStruct((NP, D // 2), jnp.float32)],
)

_finalize = pl.pallas_call(
    _finalize_body,
    grid=(NP // _BM,),
    in_specs=[
        pl.BlockSpec((NC, 2, _BM, D // 2), lambda i: (0, 0, i, 0)),
        pl.BlockSpec((1, NW, 2, _BM), lambda i: (i, 0, 0, 0)),
    ],
    out_specs=pl.BlockSpec((_BM, D), lambda i: (i, 0)),
    out_shape=jax.ShapeDtypeStruct((NP, D), jnp.float32),
)


def kernel(x, hyperedge_index, W):
    he = hyperedge_index.reshape(2, NW, CH, K)
    xp = jnp.pad(x, ((0, NP - N), (0, 0)))
    degs = _degrees_sc(he)
    sc0, sc1 = _matmul_scale(xp, W, degs)
    (acc,) = _aggregate_sc(sc0, sc1, he)
    return _finalize(acc, degs)[:N]


# speedup vs baseline: 17.1712x; 1.4845x over previous
import jax
import jax.numpy as jnp
from jax import lax
from jax.experimental import pallas as pl
from jax.experimental.pallas import tpu as pltpu
from jax.experimental.pallas import tpu_sc as plsc
from jax._src.pallas import mpmd as _mpmd
from jax._src.pallas import core as _pallas_core

N = 10000
NP = 10240
E = 320000
D = 128
NC = 2
NS = 16
NW = NC * NS
K = 128
CH = 79
T = CH * K
EPAD = NW * T - E
RPW = NP // NS
L = 16
NB = 8
DB = NP // 1280

_vmesh = plsc.VectorSubcoreMesh(core_axis_name="core", subcore_axis_name="subcore")
_smesh = plsc.ScalarSubcoreMesh(axis_name="core", num_cores=NC)


def _tec_vmem(shape, dtype):
    return _pallas_core.CoreMemorySpace(pltpu.VMEM, _vmesh)(shape, dtype)


def _deg_tec(he_hbm, out_hbm, idx_v, hist_v, cnt_v):
    c = lax.axis_index("core")
    s = lax.axis_index("subcore")
    wid = s * NC + c

    iota = lax.iota(jnp.int32, L)
    bank = lax.bitwise_and(iota, NB - 1)
    m_lo = iota < NB
    m_hi = jnp.logical_not(m_lo)
    ones = jnp.ones((L,), jnp.float32)

    @pl.loop(0, NP * NB, step=L)
    def _(i):
        hist_v[pl.ds(i, L)] = jnp.zeros((L,), jnp.float32)

    for which in range(2):
        pltpu.sync_copy(he_hbm.at[which, wid], idx_v)

        @pl.loop(0, CH)
        def _(j):
            @pl.loop(0, K, step=L)
            def _(m):
                v = idx_v[j, pl.ds(m, L)]
                flat = v * NB + bank
                plsc.addupdate_scatter(hist_v, [flat], ones, mask=m_lo)
                plsc.addupdate_scatter(hist_v, [flat], ones, mask=m_hi)

        @pl.loop(0, NP, step=L)
        def _(b):
            base = (b + iota) * NB
            acc = plsc.load_gather(hist_v, [base])
            for q in range(1, NB):
                acc = acc + plsc.load_gather(hist_v, [base + q])
            if which == 0:
                cnt_v[pl.ds(b, L)] = acc
            else:
                cnt_v[pl.ds(b, L)] = acc - cnt_v[pl.ds(b, L)]

        @pl.loop(0, DB)
        def _(t):
            pltpu.sync_copy(cnt_v.at[pl.ds(t * 1280, 1280)],
                            out_hbm.at[t, wid, which])


_degrees_sc = pl.kernel(
    _deg_tec,
    out_type=jax.ShapeDtypeStruct((DB, NW, 2, 1280), jnp.float32),
    mesh=_vmesh,
    compiler_params=pltpu.CompilerParams(needs_layout_passes=False),
    scratch_types=[
        pltpu.VMEM((CH, K), jnp.int32),
        pltpu.VMEM((NP * NB,), jnp.float32),
        pltpu.VMEM((NP,), jnp.float32),
    ],
)


def _agg_scs(*_refs):
    pass


def _agg_tec(scaled0_hbm, scaled1_hbm, he_hbm, out_hbm, idx0_v, idx1_v,
             rows_v, zero_v, acc_sh):
    pl.run_scoped(
        lambda sems: _agg_tec_inner(scaled0_hbm, scaled1_hbm, he_hbm,
                                    out_hbm, idx0_v, idx1_v, rows_v,
                                    zero_v, acc_sh, sems),
        [pltpu.SemaphoreType.DMA] * 4,
    )


def _agg_tec_inner(scaled0_hbm, scaled1_hbm, he_hbm, out_hbm, idx0_v, idx1_v,
                   rows_v, zero_v, acc_sh, sems):
    c = lax.axis_index("core")
    s = lax.axis_index("subcore")
    wid = s * NC + c
    zr = RPW // 5
    HD = D // 2

    @pl.loop(0, zr)
    def _(i):
        @pl.loop(0, HD, step=L)
        def _(j):
            zero_v[i, pl.ds(j, L)] = jnp.zeros((L,), jnp.float32)

    pltpu.sync_copy(he_hbm.at[0, wid], idx0_v)
    pltpu.sync_copy(he_hbm.at[1, wid], idx1_v)

    for h in range(2):
        src_hbm = scaled0_hbm if h == 0 else scaled1_hbm

        @pl.loop(0, 5)
        def _(t):
            pltpu.sync_copy(zero_v, acc_sh.at[pl.ds(s * RPW + t * zr, zr)])

        plsc.subcore_barrier()

        for t in range(4):
            pltpu.async_copy(src_hbm.at[idx1_v.at[t]], rows_v.at[t], sems[t])

        @pl.loop(0, CH - 3, step=4)
        def _(j):
            for t in range(4):
                jt = j + t
                pltpu.make_async_copy(src_hbm.at[idx1_v.at[jt]],
                                      rows_v.at[t], sems[t]).wait()
                pltpu.sync_copy(rows_v.at[t], acc_sh.at[idx0_v.at[jt]],
                                add=True)
                if t < 3:
                    pltpu.async_copy(src_hbm.at[idx1_v.at[jt + 4]],
                                     rows_v.at[t],
                                     sems[t])
                else:
                    @pl.when(jt + 4 < CH)
                    def _():
                        pltpu.async_copy(src_hbm.at[idx1_v.at[jt + 4]],
                                         rows_v.at[t], sems[t])

        for t in range(3):
            last = CH - 3 + t
            pltpu.make_async_copy(src_hbm.at[idx1_v.at[last]], rows_v.at[t],
                                  sems[t]).wait()
            pltpu.sync_copy(rows_v.at[t], acc_sh.at[idx0_v.at[last]],
                            add=True)

        plsc.subcore_barrier()
        pltpu.sync_copy(acc_sh.at[pl.ds(s * RPW, RPW)],
                        out_hbm.at[c, h, pl.ds(s * RPW, RPW)])
        plsc.subcore_barrier()


_aggregate_sc = _mpmd.mpmd_map(
    [(_smesh, _agg_scs), (_vmesh, _agg_tec)],
    out_types=[jax.ShapeDtypeStruct((NC, 2, NP, D // 2), jnp.float32)],
    scratch_types=[
        _tec_vmem((CH, K), jnp.int32),
        _tec_vmem((CH, K), jnp.int32),
        _tec_vmem((4, K, D // 2), jnp.float32),
        _tec_vmem((RPW // 5, D // 2), jnp.float32),
        pltpu.VMEM_SHARED((NP, D // 2), jnp.float32),
    ],
    compiler_params=pltpu.CompilerParams(use_tc_tiling_on_sc=False),
)


_BM = 1280


def _matmul_scale_body(x_ref, w_ref, degp_ref, o0_ref, o1_ref):
    d = jnp.sum(degp_ref[0][:, 1, :], axis=0)
    b = 1.0 / (d + 1e-5)
    bcol = jnp.broadcast_to(b.reshape(_BM, 1), (_BM, D // 2))
    acc = jnp.dot(x_ref[...], w_ref[...], preferred_element_type=jnp.float32)
    o0_ref[...] = acc[:, :D // 2] * bcol
    o1_ref[...] = acc[:, D // 2:] * bcol


def _finalize_body(acc_ref, degp_ref, o_ref):
    s0 = acc_ref[0, 0] + acc_ref[1, 0]
    s1 = acc_ref[0, 1] + acc_ref[1, 1]
    ssum = jnp.concatenate([s0, s1], axis=1)
    d = jnp.sum(degp_ref[0][:, 0, :], axis=0)
    a = jnp.where(d > 0, lax.rsqrt(d), 0.0)
    acol = jnp.broadcast_to(a.reshape(_BM, 1), (_BM, D))
    o_ref[...] = ssum * acol


_matmul_scale = pl.pallas_call(
    _matmul_scale_body,
    grid=(NP // _BM,),
    in_specs=[
        pl.BlockSpec((_BM, D), lambda i: (i, 0)),
        pl.BlockSpec((D, D), lambda i: (0, 0)),
        pl.BlockSpec((1, NW, 2, _BM), lambda i: (i, 0, 0, 0)),
    ],
    out_specs=[pl.BlockSpec((_BM, D // 2), lambda i: (i, 0)),
               pl.BlockSpec((_BM, D // 2), lambda i: (i, 0))],
    out_shape=[jax.ShapeDtypeStruct((NP, D // 2), jnp.float32),
               jax.ShapeDtypeStruct((NP, D // 2), jnp.float32)],
)

_finalize = pl.pallas_call(
    _finalize_body,
    grid=(NP // _BM,),
    in_specs=[
        pl.BlockSpec((NC, 2, _BM, D // 2), lambda i: (0, 0, i, 0)),
        pl.BlockSpec((1, NW, 2, _BM), lambda i: (i, 0, 0, 0)),
    ],
    out_specs=pl.BlockSpec((_BM, D), lambda i: (i, 0)),
    out_shape=jax.ShapeDtypeStruct((NP, D), jnp.float32),
)


def kernel(x, hyperedge_index, W):
    pad = N + (jnp.arange(EPAD, dtype=jnp.int32) % (NP - N))
    he = jnp.concatenate(
        [hyperedge_index, jnp.stack([pad, pad])], axis=1
    ).reshape(2, NW, CH, K)
    xp = jnp.pad(x, ((0, NP - N), (0, 0)))
    degs = _degrees_sc(he)
    sc0, sc1 = _matmul_scale(xp, W, degs)
    (acc,) = _aggregate_sc(sc0, sc1, he)
    return _finalize(acc, degs)[:N]


# speedup vs baseline: 19.8679x; 1.1570x over previous
import jax
import jax.numpy as jnp
from jax import lax
from jax.experimental import pallas as pl
from jax.experimental.pallas import tpu as pltpu
from jax.experimental.pallas import tpu_sc as plsc
from jax._src.pallas import mpmd as _mpmd
from jax._src.pallas import core as _pallas_core

N = 10000
NP = 10240
E = 320000
D = 128
NC = 2
NS = 16
NW = NC * NS
K = 128
CH = 79
T = CH * K
EPAD = NW * T - E
RPW = NP // NS
L = 16
NB = 8
DB = NP // 1280

_vmesh = plsc.VectorSubcoreMesh(core_axis_name="core", subcore_axis_name="subcore")
_smesh = plsc.ScalarSubcoreMesh(axis_name="core", num_cores=NC)


def _tec_vmem(shape, dtype):
    return _pallas_core.CoreMemorySpace(pltpu.VMEM, _vmesh)(shape, dtype)


def _deg_tec(he_hbm, out_hbm, idx_v, hist_v, cnt_v):
    c = lax.axis_index("core")
    s = lax.axis_index("subcore")
    wid = s * NC + c

    iota = lax.iota(jnp.int32, L)
    bankoff = lax.bitwise_and(iota, NB - 1) * NP
    m_lo = iota < NB
    m_hi = jnp.logical_not(m_lo)
    ones = jnp.ones((L,), jnp.float32)

    @pl.loop(0, NP * NB, step=L, unroll=4)
    def _(i):
        hist_v[pl.ds(i, L)] = jnp.zeros((L,), jnp.float32)

    for which in range(2):
        pltpu.sync_copy(he_hbm.at[which, wid], idx_v)

        @pl.loop(0, CH)
        def _(j):
            @pl.loop(0, K, step=L, unroll=4)
            def _(m):
                v = idx_v[j, pl.ds(m, L)]
                flat = v + bankoff
                plsc.addupdate_scatter(hist_v, [flat], ones, mask=m_lo)
                plsc.addupdate_scatter(hist_v, [flat], ones, mask=m_hi)

        @pl.loop(0, NP, step=L, unroll=4)
        def _(b):
            acc = hist_v[pl.ds(b, L)]
            for q in range(1, NB):
                acc = acc + hist_v[pl.ds(q * NP + b, L)]
            if which == 0:
                cnt_v[pl.ds(b, L)] = acc
            else:
                cnt_v[pl.ds(b, L)] = acc - cnt_v[pl.ds(b, L)]

        @pl.loop(0, DB)
        def _(t):
            pltpu.sync_copy(cnt_v.at[pl.ds(t * 1280, 1280)],
                            out_hbm.at[t, wid, which])


_degrees_sc = pl.kernel(
    _deg_tec,
    out_type=jax.ShapeDtypeStruct((DB, NW, 2, 1280), jnp.float32),
    mesh=_vmesh,
    compiler_params=pltpu.CompilerParams(needs_layout_passes=False),
    scratch_types=[
        pltpu.VMEM((CH, K), jnp.int32),
        pltpu.VMEM((NP * NB,), jnp.float32),
        pltpu.VMEM((NP,), jnp.float32),
    ],
)


def _agg_scs(*_refs):
    pass


def _agg_tec(scaled0_hbm, scaled1_hbm, he_hbm, out_hbm, idx0_v, idx1_v,
             rows_v, zero_v, acc_sh):
    pl.run_scoped(
        lambda sems: _agg_tec_inner(scaled0_hbm, scaled1_hbm, he_hbm,
                                    out_hbm, idx0_v, idx1_v, rows_v,
                                    zero_v, acc_sh, sems),
        [pltpu.SemaphoreType.DMA] * 4,
    )


def _agg_tec_inner(scaled0_hbm, scaled1_hbm, he_hbm, out_hbm, idx0_v, idx1_v,
                   rows_v, zero_v, acc_sh, sems):
    c = lax.axis_index("core")
    s = lax.axis_index("subcore")
    wid = s * NC + c
    zr = RPW // 5
    HD = D // 2

    @pl.loop(0, zr)
    def _(i):
        @pl.loop(0, HD, step=L)
        def _(j):
            zero_v[i, pl.ds(j, L)] = jnp.zeros((L,), jnp.float32)

    pltpu.sync_copy(he_hbm.at[0, wid], idx0_v)
    pltpu.sync_copy(he_hbm.at[1, wid], idx1_v)

    for h in range(2):
        src_hbm = scaled0_hbm if h == 0 else scaled1_hbm

        @pl.loop(0, 5)
        def _(t):
            pltpu.sync_copy(zero_v, acc_sh.at[pl.ds(s * RPW + t * zr, zr)])

        plsc.subcore_barrier()

        for t in range(4):
            pltpu.async_copy(src_hbm.at[idx1_v.at[t]], rows_v.at[t], sems[t])

        @pl.loop(0, CH - 3, step=4)
        def _(j):
            for t in range(4):
                jt = j + t
                pltpu.make_async_copy(src_hbm.at[idx1_v.at[jt]],
                                      rows_v.at[t], sems[t]).wait()
                pltpu.sync_copy(rows_v.at[t], acc_sh.at[idx0_v.at[jt]],
                                add=True)
                if t < 3:
                    pltpu.async_copy(src_hbm.at[idx1_v.at[jt + 4]],
                                     rows_v.at[t],
                                     sems[t])
                else:
                    @pl.when(jt + 4 < CH)
                    def _():
                        pltpu.async_copy(src_hbm.at[idx1_v.at[jt + 4]],
                                         rows_v.at[t], sems[t])

        for t in range(3):
            last = CH - 3 + t
            pltpu.make_async_copy(src_hbm.at[idx1_v.at[last]], rows_v.at[t],
                                  sems[t]).wait()
            pltpu.sync_copy(rows_v.at[t], acc_sh.at[idx0_v.at[last]],
                            add=True)

        plsc.subcore_barrier()
        pltpu.sync_copy(acc_sh.at[pl.ds(s * RPW, RPW)],
                        out_hbm.at[c, h, pl.ds(s * RPW, RPW)])
        plsc.subcore_barrier()


_aggregate_sc = _mpmd.mpmd_map(
    [(_smesh, _agg_scs), (_vmesh, _agg_tec)],
    out_types=[jax.ShapeDtypeStruct((NC, 2, NP, D // 2), jnp.float32)],
    scratch_types=[
        _tec_vmem((CH, K), jnp.int32),
        _tec_vmem((CH, K), jnp.int32),
        _tec_vmem((4, K, D // 2), jnp.float32),
        _tec_vmem((RPW // 5, D // 2), jnp.float32),
        pltpu.VMEM_SHARED((NP, D // 2), jnp.float32),
    ],
    compiler_params=pltpu.CompilerParams(use_tc_tiling_on_sc=False),
)


_BM = 1280


def _matmul_body(x_ref, w_ref, o_ref):
    o_ref[...] = jnp.dot(x_ref[...], w_ref[...],
                         preferred_element_type=jnp.float32)


def _scale_body(mm_ref, degp_ref, o0_ref, o1_ref):
    d = jnp.sum(degp_ref[0][:, 1, :], axis=0)
    b = 1.0 / (d + 1e-5)
    bcol = jnp.broadcast_to(b.reshape(_BM, 1), (_BM, D // 2))
    acc = mm_ref[...]
    o0_ref[...] = acc[:, :D // 2] * bcol
    o1_ref[...] = acc[:, D // 2:] * bcol


def _finalize_body(acc_ref, degp_ref, o_ref):
    s0 = acc_ref[0, 0] + acc_ref[1, 0]
    s1 = acc_ref[0, 1] + acc_ref[1, 1]
    ssum = jnp.concatenate([s0, s1], axis=1)
    d = jnp.sum(degp_ref[0][:, 0, :], axis=0)
    a = jnp.where(d > 0, lax.rsqrt(d), 0.0)
    acol = jnp.broadcast_to(a.reshape(_BM, 1), (_BM, D))
    o_ref[...] = ssum * acol


_matmul = pl.pallas_call(
    _matmul_body,
    grid=(NP // _BM,),
    in_specs=[
        pl.BlockSpec((_BM, D), lambda i: (i, 0)),
        pl.BlockSpec((D, D), lambda i: (0, 0)),
    ],
    out_specs=pl.BlockSpec((_BM, D), lambda i: (i, 0)),
    out_shape=jax.ShapeDtypeStruct((NP, D), jnp.float32),
)

_scale = pl.pallas_call(
    _scale_body,
    grid=(NP // _BM,),
    in_specs=[
        pl.BlockSpec((_BM, D), lambda i: (i, 0)),
        pl.BlockSpec((1, NW, 2, _BM), lambda i: (i, 0, 0, 0)),
    ],
    out_specs=[pl.BlockSpec((_BM, D // 2), lambda i: (i, 0)),
               pl.BlockSpec((_BM, D // 2), lambda i: (i, 0))],
    out_shape=[jax.ShapeDtypeStruct((NP, D // 2), jnp.float32),
               jax.ShapeDtypeStruct((NP, D // 2), jnp.float32)],
)

_finalize = pl.pallas_call(
    _finalize_body,
    grid=(NP // _BM,),
    in_specs=[
        pl.BlockSpec((NC, 2, _BM, D // 2), lambda i: (0, 0, i, 0)),
        pl.BlockSpec((1, NW, 2, _BM), lambda i: (i, 0, 0, 0)),
    ],
    out_specs=pl.BlockSpec((_BM, D), lambda i: (i, 0)),
    out_shape=jax.ShapeDtypeStruct((NP, D), jnp.float32),
)


def kernel(x, hyperedge_index, W):
    pad = N + (jnp.arange(EPAD, dtype=jnp.int32) % (NP - N))
    he = jnp.concatenate(
        [hyperedge_index, jnp.stack([pad, pad])], axis=1
    ).reshape(2, NW, CH, K)
    xp = jnp.pad(x, ((0, NP - N), (0, 0)))
    degs = _degrees_sc(he)
    mm = _matmul(xp, W)
    sc0, sc1 = _scale(mm, degs)
    (acc,) = _aggregate_sc(sc0, sc1, he)
    return _finalize(acc, degs)[:N]


# speedup vs baseline: 19.8879x; 1.0010x over previous
import jax
import jax.numpy as jnp
from jax import lax
from jax.experimental import pallas as pl
from jax.experimental.pallas import tpu as pltpu
from jax.experimental.pallas import tpu_sc as plsc
from jax._src.pallas import mpmd as _mpmd
from jax._src.pallas import core as _pallas_core

N = 10000
NP = 10240
E = 320000
D = 128
NC = 2
NS = 16
NW = NC * NS
K = 128
CH = 79
T = CH * K
EPAD = NW * T - E
RPW = NP // NS
L = 16
NB = 8
DB = NP // 1280

_vmesh = plsc.VectorSubcoreMesh(core_axis_name="core", subcore_axis_name="subcore")
_smesh = plsc.ScalarSubcoreMesh(axis_name="core", num_cores=NC)


def _tec_vmem(shape, dtype):
    return _pallas_core.CoreMemorySpace(pltpu.VMEM, _vmesh)(shape, dtype)


def _deg_tec(he_hbm, out_hbm, idx_v, hist_v, cnt_v):
    c = lax.axis_index("core")
    s = lax.axis_index("subcore")
    wid = s * NC + c

    iota = lax.iota(jnp.int32, L)
    bankoff = lax.bitwise_and(iota, NB - 1) * NP
    m_lo = iota < NB
    m_hi = jnp.logical_not(m_lo)
    ones = jnp.ones((L,), jnp.float32)

    @pl.loop(0, NP * NB, step=L, unroll=4)
    def _(i):
        hist_v[pl.ds(i, L)] = jnp.zeros((L,), jnp.float32)

    for which in range(2):
        pltpu.sync_copy(he_hbm.at[which, wid], idx_v)

        @pl.loop(0, CH)
        def _(j):
            @pl.loop(0, K, step=L, unroll=4)
            def _(m):
                v = idx_v[j, pl.ds(m, L)]
                flat = v + bankoff
                plsc.addupdate_scatter(hist_v, [flat], ones, mask=m_lo)
                plsc.addupdate_scatter(hist_v, [flat], ones, mask=m_hi)

        @pl.loop(0, NP, step=L, unroll=4)
        def _(b):
            acc = hist_v[pl.ds(b, L)]
            for q in range(1, NB):
                acc = acc + hist_v[pl.ds(q * NP + b, L)]
            if which == 0:
                cnt_v[pl.ds(b, L)] = acc
            else:
                cnt_v[pl.ds(b, L)] = acc - cnt_v[pl.ds(b, L)]

        @pl.loop(0, DB)
        def _(t):
            pltpu.sync_copy(cnt_v.at[pl.ds(t * 1280, 1280)],
                            out_hbm.at[t, wid, which])


_degrees_sc = pl.kernel(
    _deg_tec,
    out_type=jax.ShapeDtypeStruct((DB, NW, 2, 1280), jnp.float32),
    mesh=_vmesh,
    compiler_params=pltpu.CompilerParams(needs_layout_passes=False),
    scratch_types=[
        pltpu.VMEM((CH, K), jnp.int32),
        pltpu.VMEM((NP * NB,), jnp.float32),
        pltpu.VMEM((NP,), jnp.float32),
    ],
)


def _agg_scs(*_refs):
    pass


def _agg_tec(scaled0_hbm, scaled1_hbm, he_hbm, out_hbm, idx0_v, idx1_v,
             rows_v, zero_v, acc_sh):
    pl.run_scoped(
        lambda sems: _agg_tec_inner(scaled0_hbm, scaled1_hbm, he_hbm,
                                    out_hbm, idx0_v, idx1_v, rows_v,
                                    zero_v, acc_sh, sems),
        [pltpu.SemaphoreType.DMA] * 6,
    )


def _agg_tec_inner(scaled0_hbm, scaled1_hbm, he_hbm, out_hbm, idx0_v, idx1_v,
                   rows_v, zero_v, acc_sh, sems):
    c = lax.axis_index("core")
    s = lax.axis_index("subcore")
    wid = s * NC + c
    zr = RPW // 5
    HD = D // 2

    @pl.loop(0, zr)
    def _(i):
        @pl.loop(0, HD, step=L)
        def _(j):
            zero_v[i, pl.ds(j, L)] = jnp.zeros((L,), jnp.float32)

    pltpu.sync_copy(he_hbm.at[0, wid], idx0_v)
    pltpu.sync_copy(he_hbm.at[1, wid], idx1_v)

    for h in range(2):
        src_hbm = scaled0_hbm if h == 0 else scaled1_hbm

        @pl.loop(0, 5)
        def _(t):
            pltpu.sync_copy(zero_v, acc_sh.at[pl.ds(s * RPW + t * zr, zr)])

        plsc.subcore_barrier()

        for t in range(6):
            pltpu.async_copy(src_hbm.at[idx1_v.at[t]], rows_v.at[t], sems[t])

        @pl.loop(0, 73, step=6)
        def _(j):
            for t in range(6):
                jt = j + t
                pltpu.make_async_copy(src_hbm.at[idx1_v.at[jt]],
                                      rows_v.at[t], sems[t]).wait()
                pltpu.sync_copy(rows_v.at[t], acc_sh.at[idx0_v.at[jt]],
                                add=True)

                @pl.when(jt + 6 < CH)
                def _():
                    pltpu.async_copy(src_hbm.at[idx1_v.at[jt + 6]],
                                     rows_v.at[t], sems[t])

        last = CH - 1
        pltpu.make_async_copy(src_hbm.at[idx1_v.at[last]], rows_v.at[0],
                              sems[0]).wait()
        pltpu.sync_copy(rows_v.at[0], acc_sh.at[idx0_v.at[last]], add=True)

        plsc.subcore_barrier()
        pltpu.sync_copy(acc_sh.at[pl.ds(s * RPW, RPW)],
                        out_hbm.at[c, h, pl.ds(s * RPW, RPW)])
        plsc.subcore_barrier()


_aggregate_sc = _mpmd.mpmd_map(
    [(_smesh, _agg_scs), (_vmesh, _agg_tec)],
    out_types=[jax.ShapeDtypeStruct((NC, 2, NP, D // 2), jnp.float32)],
    scratch_types=[
        _tec_vmem((CH, K), jnp.int32),
        _tec_vmem((CH, K), jnp.int32),
        _tec_vmem((6, K, D // 2), jnp.float32),
        _tec_vmem((RPW // 5, D // 2), jnp.float32),
        pltpu.VMEM_SHARED((NP, D // 2), jnp.float32),
    ],
    compiler_params=pltpu.CompilerParams(use_tc_tiling_on_sc=False),
)


_BM = 1280


def _matmul_body(x_ref, w_ref, o_ref):
    o_ref[...] = jnp.dot(x_ref[...], w_ref[...],
                         preferred_element_type=jnp.float32)


def _scale_body(mm_ref, degp_ref, o0_ref, o1_ref):
    d = jnp.sum(degp_ref[0][:, 1, :], axis=0)
    b = 1.0 / (d + 1e-5)
    bcol = jnp.broadcast_to(b.reshape(_BM, 1), (_BM, D // 2))
    acc = mm_ref[...]
    o0_ref[...] = acc[:, :D // 2] * bcol
    o1_ref[...] = acc[:, D // 2:] * bcol


def _finalize_body(acc_ref, degp_ref, o_ref):
    s0 = acc_ref[0, 0] + acc_ref[1, 0]
    s1 = acc_ref[0, 1] + acc_ref[1, 1]
    ssum = jnp.concatenate([s0, s1], axis=1)
    d = jnp.sum(degp_ref[0][:, 0, :], axis=0)
    a = jnp.where(d > 0, lax.rsqrt(d), 0.0)
    acol = jnp.broadcast_to(a.reshape(_BM, 1), (_BM, D))
    o_ref[...] = ssum * acol


_matmul = pl.pallas_call(
    _matmul_body,
    grid=(NP // _BM,),
    in_specs=[
        pl.BlockSpec((_BM, D), lambda i: (i, 0)),
        pl.BlockSpec((D, D), lambda i: (0, 0)),
    ],
    out_specs=pl.BlockSpec((_BM, D), lambda i: (i, 0)),
    out_shape=jax.ShapeDtypeStruct((NP, D), jnp.float32),
)

_scale = pl.pallas_call(
    _scale_body,
    grid=(NP // _BM,),
    in_specs=[
        pl.BlockSpec((_BM, D), lambda i: (i, 0)),
        pl.BlockSpec((1, NW, 2, _BM), lambda i: (i, 0, 0, 0)),
    ],
    out_specs=[pl.BlockSpec((_BM, D // 2), lambda i: (i, 0)),
               pl.BlockSpec((_BM, D // 2), lambda i: (i, 0))],
    out_shape=[jax.ShapeDtypeStruct((NP, D // 2), jnp.float32),
               jax.ShapeDtypeStruct((NP, D // 2), jnp.float32)],
)

_finalize = pl.pallas_call(
    _finalize_body,
    grid=(NP // _BM,),
    in_specs=[
        pl.BlockSpec((NC, 2, _BM, D // 2), lambda i: (0, 0, i, 0)),
        pl.BlockSpec((1, NW, 2, _BM), lambda i: (i, 0, 0, 0)),
    ],
    out_specs=pl.BlockSpec((_BM, D), lambda i: (i, 0)),
    out_shape=jax.ShapeDtypeStruct((NP, D), jnp.float32),
)


def kernel(x, hyperedge_index, W):
    pad = N + (jnp.arange(EPAD, dtype=jnp.int32) % (NP - N))
    he = jnp.concatenate(
        [hyperedge_index, jnp.stack([pad, pad])], axis=1
    ).reshape(2, NW, CH, K)
    xp = jnp.pad(x, ((0, NP - N), (0, 0)))
    degs = _degrees_sc(he)
    mm = _matmul(xp, W)
    sc0, sc1 = _scale(mm, degs)
    (acc,) = _aggregate_sc(sc0, sc1, he)
    return _finalize(acc, degs)[:N]


# speedup vs baseline: 21.8683x; 1.0996x over previous
import jax
import jax.numpy as jnp
from jax import lax
from jax.experimental import pallas as pl
from jax.experimental.pallas import tpu as pltpu
from jax.experimental.pallas import tpu_sc as plsc
from jax._src.pallas import mpmd as _mpmd
from jax._src.pallas import core as _pallas_core

N = 10000
NP = 10240
E = 320000
D = 128
NC = 2
NS = 16
NW = NC * NS
K = 128
CH = 79
T = CH * K
EPAD = NW * T - E
RPW = NP // NS
L = 16
NB = 8
DB = NP // 1280

_vmesh = plsc.VectorSubcoreMesh(core_axis_name="core", subcore_axis_name="subcore")
_smesh = plsc.ScalarSubcoreMesh(axis_name="core", num_cores=NC)


def _tec_vmem(shape, dtype):
    return _pallas_core.CoreMemorySpace(pltpu.VMEM, _vmesh)(shape, dtype)


def _deg_tec(he_hbm, out_hbm, idx_v, hist_v, cnt_v):
    c = lax.axis_index("core")
    s = lax.axis_index("subcore")
    wid = s * NC + c

    iota = lax.iota(jnp.int32, L)
    bankoff = lax.bitwise_and(iota, NB - 1) * NP
    m_lo = iota < NB
    m_hi = jnp.logical_not(m_lo)
    ones = jnp.ones((L,), jnp.float32)

    @pl.loop(0, NP * NB, step=L, unroll=4)
    def _(i):
        hist_v[pl.ds(i, L)] = jnp.zeros((L,), jnp.float32)

    for which in range(2):
        pltpu.sync_copy(he_hbm.at[which, wid], idx_v)

        @pl.loop(0, CH)
        def _(j):
            @pl.loop(0, K, step=L, unroll=4)
            def _(m):
                v = idx_v[j, pl.ds(m, L)]
                flat = v + bankoff
                plsc.addupdate_scatter(hist_v, [flat], ones, mask=m_lo)
                plsc.addupdate_scatter(hist_v, [flat], ones, mask=m_hi)

        @pl.loop(0, NP, step=L, unroll=4)
        def _(b):
            acc = hist_v[pl.ds(b, L)]
            for q in range(1, NB):
                acc = acc + hist_v[pl.ds(q * NP + b, L)]
            if which == 0:
                cnt_v[pl.ds(b, L)] = acc
            else:
                cnt_v[pl.ds(b, L)] = acc - cnt_v[pl.ds(b, L)]

        @pl.loop(0, DB)
        def _(t):
            pltpu.sync_copy(cnt_v.at[pl.ds(t * 1280, 1280)],
                            out_hbm.at[t, wid, which])


_degrees_sc = pl.kernel(
    _deg_tec,
    out_type=jax.ShapeDtypeStruct((DB, NW, 2, 1280), jnp.float32),
    mesh=_vmesh,
    compiler_params=pltpu.CompilerParams(needs_layout_passes=False),
    scratch_types=[
        pltpu.VMEM((CH, K), jnp.int32),
        pltpu.VMEM((NP * NB,), jnp.float32),
        pltpu.VMEM((NP,), jnp.float32),
    ],
)


def _agg_scs(*_refs):
    pass


def _agg_tec(scaled0_hbm, scaled1_hbm, he_hbm, out_hbm, idx0_v, idx1_v,
             rows_v, zero_v, acc_sh):
    pl.run_scoped(
        lambda sems: _agg_tec_inner(scaled0_hbm, scaled1_hbm, he_hbm,
                                    out_hbm, idx0_v, idx1_v, rows_v,
                                    zero_v, acc_sh, sems),
        [pltpu.SemaphoreType.DMA] * 6,
    )


def _agg_tec_inner(scaled0_hbm, scaled1_hbm, he_hbm, out_hbm, idx0_v, idx1_v,
                   rows_v, zero_v, acc_sh, sems):
    c = lax.axis_index("core")
    s = lax.axis_index("subcore")
    wid = s * NC + c
    zr = RPW // 5
    HD = D // 2

    @pl.loop(0, zr)
    def _(i):
        @pl.loop(0, HD, step=L)
        def _(j):
            zero_v[i, pl.ds(j, L)] = jnp.zeros((L,), jnp.float32)

    pltpu.sync_copy(he_hbm.at[0, wid], idx0_v)
    pltpu.sync_copy(he_hbm.at[1, wid], idx1_v)

    for h in range(2):
        src_hbm = scaled0_hbm if h == 0 else scaled1_hbm

        @pl.loop(0, 5)
        def _(t):
            pltpu.sync_copy(zero_v, acc_sh.at[pl.ds(s * RPW + t * zr, zr)])

        plsc.subcore_barrier()

        for t in range(6):
            pltpu.async_copy(src_hbm.at[idx1_v.at[t]], rows_v.at[t], sems[t])

        @pl.loop(0, 73, step=6)
        def _(j):
            for t in range(6):
                jt = j + t
                pltpu.make_async_copy(src_hbm.at[idx1_v.at[jt]],
                                      rows_v.at[t], sems[t]).wait()
                pltpu.sync_copy(rows_v.at[t], acc_sh.at[idx0_v.at[jt]],
                                add=True)

                @pl.when(jt + 6 < CH)
                def _():
                    pltpu.async_copy(src_hbm.at[idx1_v.at[jt + 6]],
                                     rows_v.at[t], sems[t])

        last = CH - 1
        pltpu.make_async_copy(src_hbm.at[idx1_v.at[last]], rows_v.at[0],
                              sems[0]).wait()
        pltpu.sync_copy(rows_v.at[0], acc_sh.at[idx0_v.at[last]], add=True)

        plsc.subcore_barrier()
        pltpu.sync_copy(acc_sh.at[pl.ds(s * RPW, RPW)],
                        out_hbm.at[c, pl.ds(s * RPW, RPW),
                                   pl.ds(h * (D // 2), D // 2)])
        plsc.subcore_barrier()


_aggregate_sc = _mpmd.mpmd_map(
    [(_smesh, _agg_scs), (_vmesh, _agg_tec)],
    out_types=[jax.ShapeDtypeStruct((NC, NP, D), jnp.float32)],
    scratch_types=[
        _tec_vmem((CH, K), jnp.int32),
        _tec_vmem((CH, K), jnp.int32),
        _tec_vmem((6, K, D // 2), jnp.float32),
        _tec_vmem((RPW // 5, D // 2), jnp.float32),
        pltpu.VMEM_SHARED((NP, D // 2), jnp.float32),
    ],
    compiler_params=pltpu.CompilerParams(use_tc_tiling_on_sc=False),
)


_BM = 1280


def _matmul_body(x_ref, w_ref, o_ref):
    o_ref[...] = jnp.dot(x_ref[...], w_ref[...],
                         preferred_element_type=jnp.float32)


def _scale_body(mm_ref, degp_ref, o0_ref, o1_ref):
    d = jnp.sum(degp_ref[0][:, 1, :], axis=0)
    b = 1.0 / (d + 1e-5)
    bcol = jnp.broadcast_to(b.reshape(_BM, 1), (_BM, D // 2))
    acc = mm_ref[...]
    o0_ref[...] = acc[:, :D // 2] * bcol
    o1_ref[...] = acc[:, D // 2:] * bcol


def _finalize_body(acc_ref, degp_ref, o_ref):
    ssum = acc_ref[0] + acc_ref[1]
    d = jnp.sum(degp_ref[0][:, 0, :], axis=0)
    a = jnp.where(d > 0, lax.rsqrt(d), 0.0)
    acol = jnp.broadcast_to(a.reshape(_BM, 1), (_BM, D))
    o_ref[...] = ssum * acol


_matmul = pl.pallas_call(
    _matmul_body,
    grid=(NP // _BM,),
    in_specs=[
        pl.BlockSpec((_BM, D), lambda i: (i, 0)),
        pl.BlockSpec((D, D), lambda i: (0, 0)),
    ],
    out_specs=pl.BlockSpec((_BM, D), lambda i: (i, 0)),
    out_shape=jax.ShapeDtypeStruct((NP, D), jnp.float32),
)

_scale = pl.pallas_call(
    _scale_body,
    grid=(NP // _BM,),
    in_specs=[
        pl.BlockSpec((_BM, D), lambda i: (i, 0)),
        pl.BlockSpec((1, NW, 2, _BM), lambda i: (i, 0, 0, 0)),
    ],
    out_specs=[pl.BlockSpec((_BM, D // 2), lambda i: (i, 0)),
               pl.BlockSpec((_BM, D // 2), lambda i: (i, 0))],
    out_shape=[jax.ShapeDtypeStruct((NP, D // 2), jnp.float32),
               jax.ShapeDtypeStruct((NP, D // 2), jnp.float32)],
)

_finalize = pl.pallas_call(
    _finalize_body,
    grid=(NP // _BM,),
    in_specs=[
        pl.BlockSpec((NC, _BM, D), lambda i: (0, i, 0)),
        pl.BlockSpec((1, NW, 2, _BM), lambda i: (i, 0, 0, 0)),
    ],
    out_specs=pl.BlockSpec((_BM, D), lambda i: (i, 0)),
    out_shape=jax.ShapeDtypeStruct((NP, D), jnp.float32),
)


def kernel(x, hyperedge_index, W):
    pad = N + (jnp.arange(EPAD, dtype=jnp.int32) % (NP - N))
    he = jnp.concatenate(
        [hyperedge_index, jnp.stack([pad, pad])], axis=1
    ).reshape(2, NW, CH, K)
    xp = jnp.pad(x, ((0, NP - N), (0, 0)))
    degs = _degrees_sc(he)
    mm = _matmul(xp, W)
    sc0, sc1 = _scale(mm, degs)
    (acc,) = _aggregate_sc(sc0, sc1, he)
    return _finalize(acc, degs)[:N]


# speedup vs baseline: 22.3917x; 1.0239x over previous
import jax
import jax.numpy as jnp
from jax import lax
from jax.experimental import pallas as pl
from jax.experimental.pallas import tpu as pltpu
from jax.experimental.pallas import tpu_sc as plsc
from jax._src.pallas import mpmd as _mpmd
from jax._src.pallas import core as _pallas_core

N = 10000
NP = 10240
E = 320000
D = 128
NC = 2
NS = 16
NW = NC * NS
K = 128
CH = 79
T = CH * K
EPAD = NW * T - E
RPW = NP // NS
L = 16
NB = 8
DB = NP // 1280

_vmesh = plsc.VectorSubcoreMesh(core_axis_name="core", subcore_axis_name="subcore")
_smesh = plsc.ScalarSubcoreMesh(axis_name="core", num_cores=NC)


def _tec_vmem(shape, dtype):
    return _pallas_core.CoreMemorySpace(pltpu.VMEM, _vmesh)(shape, dtype)


def _deg_tec(he_hbm, out_hbm, idx_v, hist_v, cnt_v):
    c = lax.axis_index("core")
    s = lax.axis_index("subcore")
    wid = s * NC + c

    iota = lax.iota(jnp.int32, L)
    bankoff = lax.bitwise_and(iota, NB - 1) * NP
    m_lo = iota < NB
    m_hi = jnp.logical_not(m_lo)
    ones = jnp.ones((L,), jnp.float32)

    @pl.loop(0, NP * NB, step=L, unroll=4)
    def _(i):
        hist_v[pl.ds(i, L)] = jnp.zeros((L,), jnp.float32)

    for which in range(2):
        pltpu.sync_copy(he_hbm.at[which, wid], idx_v)

        @pl.loop(0, CH)
        def _(j):
            @pl.loop(0, K, step=L, unroll=4)
            def _(m):
                v = idx_v[j, pl.ds(m, L)]
                flat = v + bankoff
                plsc.addupdate_scatter(hist_v, [flat], ones, mask=m_lo)
                plsc.addupdate_scatter(hist_v, [flat], ones, mask=m_hi)

        @pl.loop(0, NP, step=L, unroll=4)
        def _(b):
            acc = hist_v[pl.ds(b, L)]
            for q in range(1, NB):
                acc = acc + hist_v[pl.ds(q * NP + b, L)]
            if which == 0:
                cnt_v[pl.ds(b, L)] = acc
            else:
                cnt_v[pl.ds(b, L)] = acc - cnt_v[pl.ds(b, L)]

        @pl.loop(0, DB)
        def _(t):
            pltpu.sync_copy(cnt_v.at[pl.ds(t * 1280, 1280)],
                            out_hbm.at[t, wid, which])


_degrees_sc = pl.kernel(
    _deg_tec,
    out_type=jax.ShapeDtypeStruct((DB, NW, 2, 1280), jnp.float32),
    mesh=_vmesh,
    compiler_params=pltpu.CompilerParams(needs_layout_passes=False),
    scratch_types=[
        pltpu.VMEM((CH, K), jnp.int32),
        pltpu.VMEM((NP * NB,), jnp.float32),
        pltpu.VMEM((NP,), jnp.float32),
    ],
)


def _agg_scs(*_refs):
    pass


def _agg_tec(scaled0_hbm, scaled1_hbm, he_hbm, out_hbm, idx0_v, idx1_v,
             rows_v, zero_v, acc_sh):
    pl.run_scoped(
        lambda sems: _agg_tec_inner(scaled0_hbm, scaled1_hbm, he_hbm,
                                    out_hbm, idx0_v, idx1_v, rows_v,
                                    zero_v, acc_sh, sems),
        [pltpu.SemaphoreType.DMA] * 6,
    )


def _agg_tec_inner(scaled0_hbm, scaled1_hbm, he_hbm, out_hbm, idx0_v, idx1_v,
                   rows_v, zero_v, acc_sh, sems):
    c = lax.axis_index("core")
    s = lax.axis_index("subcore")
    wid = s * NC + c
    zr = RPW // 5
    HD = D // 2

    @pl.loop(0, zr)
    def _(i):
        @pl.loop(0, HD, step=L)
        def _(j):
            zero_v[i, pl.ds(j, L)] = jnp.zeros((L,), jnp.float32)

    pltpu.sync_copy(he_hbm.at[0, wid], idx0_v)
    pltpu.sync_copy(he_hbm.at[1, wid], idx1_v)

    for h in range(2):
        src_hbm = scaled0_hbm if h == 0 else scaled1_hbm

        @pl.loop(0, 5)
        def _(t):
            pltpu.sync_copy(zero_v, acc_sh.at[pl.ds(s * RPW + t * zr, zr)])

        plsc.subcore_barrier()

        for t in range(6):
            pltpu.async_copy(src_hbm.at[idx1_v.at[t]], rows_v.at[t], sems[t])

        @pl.loop(0, 73, step=6)
        def _(j):
            for t in range(6):
                jt = j + t
                pltpu.make_async_copy(src_hbm.at[idx1_v.at[jt]],
                                      rows_v.at[t], sems[t]).wait()
                pltpu.sync_copy(rows_v.at[t], acc_sh.at[idx0_v.at[jt]],
                                add=True)

                @pl.when(jt + 6 < CH)
                def _():
                    pltpu.async_copy(src_hbm.at[idx1_v.at[jt + 6]],
                                     rows_v.at[t], sems[t])

        last = CH - 1
        pltpu.make_async_copy(src_hbm.at[idx1_v.at[last]], rows_v.at[0],
                              sems[0]).wait()
        pltpu.sync_copy(rows_v.at[0], acc_sh.at[idx0_v.at[last]], add=True)

        plsc.subcore_barrier()
        pltpu.sync_copy(acc_sh.at[pl.ds(s * RPW, RPW)],
                        out_hbm.at[c, pl.ds(s * RPW, RPW),
                                   pl.ds(h * (D // 2), D // 2)])
        plsc.subcore_barrier()


_aggregate_sc = _mpmd.mpmd_map(
    [(_smesh, _agg_scs), (_vmesh, _agg_tec)],
    out_types=[jax.ShapeDtypeStruct((NC, NP, D), jnp.float32)],
    scratch_types=[
        _tec_vmem((CH, K), jnp.int32),
        _tec_vmem((CH, K), jnp.int32),
        _tec_vmem((6, K, D // 2), jnp.float32),
        _tec_vmem((RPW // 5, D // 2), jnp.float32),
        pltpu.VMEM_SHARED((NP, D // 2), jnp.float32),
    ],
    compiler_params=pltpu.CompilerParams(use_tc_tiling_on_sc=False),
)


_BM = 1280


def _matmul_body(x_ref, w_ref, o_ref):
    o_ref[...] = jnp.dot(x_ref[...], w_ref[...],
                         preferred_element_type=jnp.float32)


def _scale_body(mm_ref, degp_ref, o0_ref, o1_ref):
    d = jnp.sum(degp_ref[0][:, 1, :], axis=0)
    b = 1.0 / (d + 1e-5)
    bcol = jnp.broadcast_to(b.reshape(_BM, 1), (_BM, D // 2))
    acc = mm_ref[...]
    o0_ref[...] = acc[:, :D // 2] * bcol
    o1_ref[...] = acc[:, D // 2:] * bcol


def _finalize_body(acc_ref, degp_ref, o_ref):
    ssum = acc_ref[0] + acc_ref[1]
    d = jnp.sum(degp_ref[0][:, 0, :], axis=0)
    a = jnp.where(d > 0, lax.rsqrt(d), 0.0)
    acol = jnp.broadcast_to(a.reshape(_BM, 1), (_BM, D))
    o_ref[...] = ssum * acol


_matmul = pl.pallas_call(
    _matmul_body,
    grid=(NP // _BM,),
    in_specs=[
        pl.BlockSpec((_BM, D), lambda i: (i, 0)),
        pl.BlockSpec((D, D), lambda i: (0, 0)),
    ],
    out_specs=pl.BlockSpec((_BM, D), lambda i: (i, 0)),
    out_shape=jax.ShapeDtypeStruct((NP, D), jnp.float32),
)

_scale = pl.pallas_call(
    _scale_body,
    grid=(NP // _BM,),
    in_specs=[
        pl.BlockSpec((_BM, D), lambda i: (i, 0)),
        pl.BlockSpec((1, NW, 2, _BM), lambda i: (i, 0, 0, 0)),
    ],
    out_specs=[pl.BlockSpec((_BM, D // 2), lambda i: (i, 0)),
               pl.BlockSpec((_BM, D // 2), lambda i: (i, 0))],
    out_shape=[jax.ShapeDtypeStruct((NP, D // 2), jnp.float32),
               jax.ShapeDtypeStruct((NP, D // 2), jnp.float32)],
)

_finalize = pl.pallas_call(
    _finalize_body,
    grid=(NP // _BM,),
    in_specs=[
        pl.BlockSpec((NC, _BM, D), lambda i: (0, i, 0)),
        pl.BlockSpec((1, NW, 2, _BM), lambda i: (i, 0, 0, 0)),
    ],
    out_specs=pl.BlockSpec((_BM, D), lambda i: (i, 0)),
    out_shape=jax.ShapeDtypeStruct((N, D), jnp.float32),
)


def kernel(x, hyperedge_index, W):
    pad = N + (jnp.arange(EPAD, dtype=jnp.int32) % (NP - N))
    he = jnp.concatenate(
        [hyperedge_index, jnp.stack([pad, pad])], axis=1
    ).reshape(2, NW, CH, K)
    degs = _degrees_sc(he)
    mm = _matmul(x, W)
    sc0, sc1 = _scale(mm, degs)
    (acc,) = _aggregate_sc(sc0, sc1, he)
    return _finalize(acc, degs)
